# baseline (speedup 1.0000x reference)
import jax
import jax.numpy as jnp
from jax import lax
from jax.experimental import pallas as pl
from jax.experimental.pallas import tpu as pltpu
from jax.experimental.pallas import tpu_sc as plsc

B = 128
N = 100000
L = 16
NC = 2
NS = 16
NW = NC * NS
RPW = B // NW
CHUNK = 20000
NCHUNKS = N // CHUNK
VECS = CHUNK // L
NEG = float("-inf")
THRESH = 0.5


def _sc_body(inp, tgt, out, tgt_v, win_v, buf0_v, buf1_v, seg_v, res_v,
             sem0, sem1):
    cid = lax.axis_index("c")
    sid = lax.axis_index("s")
    wid = sid * NC + cid
    row0 = wid * RPW
    lanes = lax.broadcasted_iota(jnp.int32, (L,), 0)

    pltpu.sync_copy(tgt, tgt_v)
    grp = pl.multiple_of((row0 // L) * L, L)
    tv = tgt_v[pl.ds(grp, L)]
    lane0 = row0 % L

    tks = []
    xts = []
    for k in range(RPW):
        tk = jnp.max(jnp.where(lanes == lane0 + k, tv, jnp.int32(-1)))
        tks.append(tk)
    for k in range(RPW):
        woff = (row0 + k) * N + (tks[k] // L) * L
        pltpu.sync_copy(inp.at[pl.ds(pl.multiple_of(woff, L), L)], win_v)
        wv = win_v[...]
        xts.append(jnp.max(jnp.where(lanes == tks[k] % L, wv, NEG)))

    base = row0 * N

    def start(g, b):
        return pltpu.async_copy(
            inp.at[pl.ds(pl.multiple_of(base + g * CHUNK, L), CHUNK)],
            buf0_v if b == 0 else buf1_v,
            sem0 if b == 0 else sem1,
        )

    handles = [None, None]
    handles[0] = start(0, 0)

    acc = jnp.float32(0.0)
    total = RPW * NCHUNKS
    for g in range(total):
        k, c = g // NCHUNKS, g % NCHUNKS
        b = g % 2
        handles[b].wait()
        if g + 1 < total:
            handles[(g + 1) % 2] = start(g + 1, (g + 1) % 2)
        if c == 0:
            carry = (
                jnp.float32(NEG),
                jnp.int32(0),
                jnp.float32(NEG),
                jnp.zeros((L,), jnp.float32),
            )
        best, bch, sec, s = carry
        bref = buf0_v if b == 0 else buf1_v

        def step(i, c2, bref=bref):
            m1, s2 = c2
            v = bref[pl.ds(pl.multiple_of(i * L, L), L)]
            return jnp.maximum(m1, v), s2 + jnp.exp(v)

        m1, s = lax.fori_loop(
            0, VECS, step,
            (jnp.full((L,), NEG, jnp.float32), s), unroll=10)
        cmax = jnp.max(m1)
        t = jnp.minimum(best, cmax)
        bch = jnp.where(cmax > best, jnp.int32(c), bch)
        carry = (jnp.maximum(best, cmax), bch, jnp.maximum(sec, t), s)

        if c == NCHUNKS - 1:
            best, bch, sec, s = carry
            xt = xts[k]
            woff = (row0 + k) * N + bch * CHUNK
            pltpu.sync_copy(
                inp.at[pl.ds(pl.multiple_of(woff, L), CHUNK)], seg_v)

            def rstep(i, c2):
                m1, m2 = c2
                v = seg_v[pl.ds(pl.multiple_of(i * L, L), L)]
                t = jnp.minimum(m1, v)
                return jnp.maximum(m1, v), jnp.maximum(m2, t)

            m1r, m2r = lax.fori_loop(
                0, VECS, rstep,
                (jnp.full((L,), NEG, jnp.float32),
                 jnp.full((L,), NEG, jnp.float32)), unroll=10)
            W1 = jnp.max(m1r)
            eq = m1r == W1
            neq = jnp.sum(jnp.where(eq, jnp.int32(1), jnp.int32(0)))
            w2 = jnp.where(neq >= 2, W1, jnp.max(jnp.where(eq, NEG, m1r)))
            W2 = jnp.maximum(w2, jnp.max(m2r))
            M1 = best
            M2 = jnp.maximum(sec, W2)
            S = jnp.sum(s)
            top1 = xt == M1
            top2 = jnp.logical_and(jnp.logical_not(top1), xt == M2)
            ev = jnp.exp(jnp.where(lanes == 0, M1, M2))
            e1 = jnp.max(jnp.where(lanes == 0, ev, NEG))
            e2 = jnp.max(jnp.where(lanes == 1, ev, NEG))
            unc = jnp.where(
                top1, jnp.float32(0.8),
                jnp.where(top2, jnp.float32(0.6), jnp.float32(0.0)))
            cer = jnp.where(top1, jnp.float32(1.0), jnp.float32(0.0))
            acc = acc + jnp.where(e1 - e2 < THRESH * S, unc, cer)

    res_v[...] = jnp.where(lanes == 0, acc, jnp.float32(0.0))
    pltpu.sync_copy(res_v, out.at[pl.ds(wid * L, L)])


_sc_call = pl.kernel(
    _sc_body,
    out_type=jax.ShapeDtypeStruct((NW * L,), jnp.float32),
    mesh=plsc.VectorSubcoreMesh(core_axis_name="c", subcore_axis_name="s"),
    scratch_types=[
        pltpu.VMEM((B,), jnp.int32),
        pltpu.VMEM((L,), jnp.float32),
        pltpu.VMEM((CHUNK,), jnp.float32),
        pltpu.VMEM((CHUNK,), jnp.float32),
        pltpu.VMEM((CHUNK,), jnp.float32),
        pltpu.VMEM((L,), jnp.float32),
        pltpu.SemaphoreType.DMA,
        pltpu.SemaphoreType.DMA,
    ],
    compiler_params=pltpu.CompilerParams(needs_layout_passes=False),
)


def _reduce_body(x_ref, o_ref):
    o_ref[...] = jnp.full((1, 1), -jnp.sum(x_ref[...]), jnp.float32)


_reduce_call = pl.pallas_call(
    _reduce_body,
    out_shape=jax.ShapeDtypeStruct((1, 1), jnp.float32),
)


def kernel(input, target):
    flat = input.reshape(B * N)
    partials = _sc_call(flat, target)
    loss = _reduce_call(partials.reshape(4, NW * L // 4))
    return loss[0, 0]


# speedup vs baseline: 1.0168x; 1.0168x over previous
import jax
import jax.numpy as jnp
from jax import lax
from jax.experimental import pallas as pl
from jax.experimental.pallas import tpu as pltpu
from jax.experimental.pallas import tpu_sc as plsc

B = 128
N = 100000
L = 16
NC = 2
NS = 16
NW = NC * NS
RPW = B // NW
CHUNK = 20000
NCHUNKS = N // CHUNK
VECS = CHUNK // L
U = 5
NEG = float("-inf")
THRESH = 0.5


def _sc_body(inp, tgt, out, tgt_v, win_v, buf0_v, buf1_v, seg_v, res_v,
             sem0, sem1):
    cid = lax.axis_index("c")
    sid = lax.axis_index("s")
    wid = sid * NC + cid
    row0 = wid * RPW
    lanes = lax.broadcasted_iota(jnp.int32, (L,), 0)

    pltpu.sync_copy(tgt, tgt_v)
    grp = pl.multiple_of((row0 // L) * L, L)
    tv = tgt_v[pl.ds(grp, L)]
    lane0 = row0 % L

    tks = []
    xts = []
    for k in range(RPW):
        tk = jnp.max(jnp.where(lanes == lane0 + k, tv, jnp.int32(-1)))
        tks.append(tk)
    for k in range(RPW):
        woff = (row0 + k) * N + (tks[k] // L) * L
        pltpu.sync_copy(inp.at[pl.ds(pl.multiple_of(woff, L), L)], win_v)
        wv = win_v[...]
        xts.append(jnp.max(jnp.where(lanes == tks[k] % L, wv, NEG)))

    base = row0 * N

    def start(g, b):
        return pltpu.async_copy(
            inp.at[pl.ds(pl.multiple_of(base + g * CHUNK, L), CHUNK)],
            buf0_v if b == 0 else buf1_v,
            sem0 if b == 0 else sem1,
        )

    handles = [None, None]
    handles[0] = start(0, 0)

    acc = jnp.float32(0.0)
    total = RPW * NCHUNKS
    for g in range(total):
        k, c = g // NCHUNKS, g % NCHUNKS
        b = g % 2
        handles[b].wait()
        if g + 1 < total:
            handles[(g + 1) % 2] = start(g + 1, (g + 1) % 2)
        if c == 0:
            carry = (
                jnp.float32(NEG),
                jnp.int32(0),
                jnp.float32(NEG),
            ) + tuple(jnp.zeros((L,), jnp.float32) for _ in range(U))
        best, bch, sec = carry[:3]
        ss = carry[3:]
        bref = buf0_v if b == 0 else buf1_v

        def step(i, c2, bref=bref):
            m1s = list(c2[:U])
            s2s = list(c2[U:])
            for j in range(U):
                off = (i * U + j) * L
                v = bref[pl.ds(pl.multiple_of(off, L), L)]
                m1s[j] = jnp.maximum(m1s[j], v)
                s2s[j] = s2s[j] + jnp.exp(v)
            return tuple(m1s) + tuple(s2s)

        res = lax.fori_loop(
            0, VECS // U, step,
            tuple(jnp.full((L,), NEG, jnp.float32) for _ in range(U)) + ss,
            unroll=2)
        m1 = res[0]
        for j in range(1, U):
            m1 = jnp.maximum(m1, res[j])
        ss = res[U:]
        cmax = jnp.max(m1)
        t = jnp.minimum(best, cmax)
        bch = jnp.where(cmax > best, jnp.int32(c), bch)
        carry = (jnp.maximum(best, cmax), bch, jnp.maximum(sec, t)) + ss

        if c == NCHUNKS - 1:
            best, bch, sec = carry[:3]
            s = carry[3]
            for j in range(1, U):
                s = s + carry[3 + j]
            xt = xts[k]
            woff = (row0 + k) * N + bch * CHUNK
            pltpu.sync_copy(
                inp.at[pl.ds(pl.multiple_of(woff, L), CHUNK)], seg_v)

            def rstep(i, c2):
                m1, m2 = c2
                v = seg_v[pl.ds(pl.multiple_of(i * L, L), L)]
                t = jnp.minimum(m1, v)
                return jnp.maximum(m1, v), jnp.maximum(m2, t)

            m1r, m2r = lax.fori_loop(
                0, VECS, rstep,
                (jnp.full((L,), NEG, jnp.float32),
                 jnp.full((L,), NEG, jnp.float32)), unroll=10)
            W1 = jnp.max(m1r)
            eq = m1r == W1
            neq = jnp.sum(jnp.where(eq, jnp.int32(1), jnp.int32(0)))
            w2 = jnp.where(neq >= 2, W1, jnp.max(jnp.where(eq, NEG, m1r)))
            W2 = jnp.maximum(w2, jnp.max(m2r))
            M1 = best
            M2 = jnp.maximum(sec, W2)
            S = jnp.sum(s)
            top1 = xt == M1
            top2 = jnp.logical_and(jnp.logical_not(top1), xt == M2)
            ev = jnp.exp(jnp.where(lanes == 0, M1, M2))
            e1 = jnp.max(jnp.where(lanes == 0, ev, NEG))
            e2 = jnp.max(jnp.where(lanes == 1, ev, NEG))
            unc = jnp.where(
                top1, jnp.float32(0.8),
                jnp.where(top2, jnp.float32(0.6), jnp.float32(0.0)))
            cer = jnp.where(top1, jnp.float32(1.0), jnp.float32(0.0))
            acc = acc + jnp.where(e1 - e2 < THRESH * S, unc, cer)

    res_v[...] = jnp.where(lanes == 0, acc, jnp.float32(0.0))
    pltpu.sync_copy(res_v, out.at[pl.ds(wid * L, L)])


_sc_call = pl.kernel(
    _sc_body,
    out_type=jax.ShapeDtypeStruct((NW * L,), jnp.float32),
    mesh=plsc.VectorSubcoreMesh(core_axis_name="c", subcore_axis_name="s"),
    scratch_types=[
        pltpu.VMEM((B,), jnp.int32),
        pltpu.VMEM((L,), jnp.float32),
        pltpu.VMEM((CHUNK,), jnp.float32),
        pltpu.VMEM((CHUNK,), jnp.float32),
        pltpu.VMEM((CHUNK,), jnp.float32),
        pltpu.VMEM((L,), jnp.float32),
        pltpu.SemaphoreType.DMA,
        pltpu.SemaphoreType.DMA,
    ],
    compiler_params=pltpu.CompilerParams(needs_layout_passes=False),
)


def _reduce_body(x_ref, o_ref):
    o_ref[...] = jnp.full((1, 1), -jnp.sum(x_ref[...]), jnp.float32)


_reduce_call = pl.pallas_call(
    _reduce_body,
    out_shape=jax.ShapeDtypeStruct((1, 1), jnp.float32),
)


def kernel(input, target):
    flat = input.reshape(B * N)
    partials = _sc_call(flat, target)
    loss = _reduce_call(partials.reshape(4, NW * L // 4))
    return loss[0, 0]
